# Initial kernel scaffold; baseline (speedup 1.0000x reference)
import jax
import jax.numpy as jnp
from jax.experimental import pallas as pl


def kernel(x, lengths, mask, emb, params):
    raise NotImplementedError("write your pallas kernel here")


# speedup vs baseline: 1.9182x; 1.9182x over previous
import functools

import jax
import jax.numpy as jnp
from jax import lax
from jax.experimental import pallas as pl
from jax.experimental.pallas import tpu as pltpu
from jax.experimental.pallas import tpu_sc as plsc

N_VOCAB = 1000
DIM = 128
B = 1024
T = 200
BT = B * T
PAD = 4
TP = T + 2 * PAD


_SC_CHUNK = 128


def _sc_gather(table, idx):
    info = plsc.get_sparse_core_info()
    nc, ns = info.num_cores, info.num_subcores
    nw = nc * ns
    per_w = BT // nw
    n_chunks = per_w // _SC_CHUNK

    mesh = plsc.VectorSubcoreMesh(core_axis_name="c", subcore_axis_name="s")

    @functools.partial(
        pl.kernel,
        mesh=mesh,
        out_type=jax.ShapeDtypeStruct((BT, DIM), jnp.float32),
        scratch_types=[
            pltpu.VMEM((per_w,), jnp.int32),
            pltpu.VMEM((_SC_CHUNK, DIM), jnp.float32),
            pltpu.VMEM((_SC_CHUNK, DIM), jnp.float32),
            pltpu.SemaphoreType.DMA,
            pltpu.SemaphoreType.DMA,
        ],
    )
    def gather_kernel(table_hbm, idx_hbm, out_hbm, idx_v, rows0, rows1, sem0, sem1):
        wid = lax.axis_index("s") * nc + lax.axis_index("c")
        base = wid * per_w
        pltpu.sync_copy(idx_hbm.at[pl.ds(base, per_w)], idx_v)

        rows = (rows0, rows1)
        sems = (sem0, sem1)

        def gather_chunk(c, buf):
            pltpu.async_copy(
                table_hbm.at[idx_v.at[pl.ds(c * _SC_CHUNK, _SC_CHUNK)]],
                rows[buf], sems[buf])

        def wait_chunk(c, buf):
            pltpu.make_async_copy(
                table_hbm.at[idx_v.at[pl.ds(c * _SC_CHUNK, _SC_CHUNK)]],
                rows[buf], sems[buf]).wait()

        def drain_chunk(c, buf):
            pltpu.sync_copy(
                rows[buf], out_hbm.at[pl.ds(base + c * _SC_CHUNK, _SC_CHUNK)])

        gather_chunk(0, 0)
        gather_chunk(1, 1)

        def body(c2, carry):
            for b in range(2):
                c = c2 * 2 + b
                wait_chunk(c, b)
                gather_chunk(c + 2, b)
                drain_chunk(c, b)
            return carry

        lax.fori_loop(0, n_chunks // 2 - 1, body, 0)
        for b in range(2):
            c = n_chunks - 2 + b
            wait_chunk(c, b)
            drain_chunk(c, b)

    return gather_kernel(table, idx)


TILE_B = 16


def _tc_body(h_ref, m_ref, dw5_ref, dw9_ref, wt_ref, bs_ref, g_ref, lb_ref,
             out_ref):
    f32 = jnp.float32
    m = m_ref[...][:, :, None]
    h = h_ref[...] * m
    zpad = jnp.zeros((TILE_B, PAD, DIM), f32)

    for i in range(4):
        hp = jnp.concatenate([zpad, h, zpad], axis=1)
        t5 = jnp.zeros((TILE_B, T, DIM), f32)
        for j in range(5):
            w = dw5_ref[i, j][None, None, :]
            t5 = t5 + hp[:, 2 + j:2 + j + T, :] * w
        t9 = jnp.zeros((TILE_B, T, DIM), f32)
        for j in range(9):
            w = dw9_ref[i, j][None, None, :]
            t9 = t9 + hp[:, j:j + T, :] * w

        y = jnp.dot(t5.reshape(TILE_B * T, DIM), wt_ref[i, 0],
                    preferred_element_type=f32)
        y = y + jnp.dot(t9.reshape(TILE_B * T, DIM), wt_ref[i, 1],
                        preferred_element_type=f32)
        y = y.reshape(TILE_B, T, DIM) + bs_ref[i][None, None, :]

        mu = jnp.mean(y, axis=-1, keepdims=True)
        d = y - mu
        var = jnp.mean(d * d, axis=-1, keepdims=True)
        y = d * lax.rsqrt(var + 1e-5)
        y = y * g_ref[i][None, None, :] + lb_ref[i][None, None, :]
        h = (h + y) * m

    out_ref[...] = h


def _tc_blocks(h0, mf, dw5, dw9, wt, bs, g, lb):
    grid = (B // TILE_B,)
    full = lambda *shape: pl.BlockSpec(shape, lambda i: (0,) * len(shape))
    return pl.pallas_call(
        _tc_body,
        grid=grid,
        in_specs=[
            pl.BlockSpec((TILE_B, T, DIM), lambda i: (i, 0, 0)),
            pl.BlockSpec((TILE_B, T), lambda i: (i, 0)),
            full(4, 5, DIM),
            full(4, 9, DIM),
            full(4, 2, DIM, DIM),
            full(4, DIM),
            full(4, DIM),
            full(4, DIM),
        ],
        out_specs=pl.BlockSpec((TILE_B, T, DIM), lambda i: (i, 0, 0)),
        out_shape=jax.ShapeDtypeStruct((B, T, DIM), jnp.float32),
    )(h0, mf, dw5, dw9, wt, bs, g, lb)


@jax.jit
def _run(x, mask, emb, dw5, dw9, wt, bs, g, lb):
    table = emb.at[0].set(0.0)
    idx = x.reshape(-1).astype(jnp.int32)
    h0 = _sc_gather(table, idx).reshape(B, T, DIM)
    mf = mask.astype(jnp.float32)
    return _tc_blocks(h0, mf, dw5, dw9, wt, bs, g, lb)


def kernel(x, lengths, mask, emb, params):
    del lengths
    dw5 = jnp.stack([blk['branches'][0]['dw'][:, 0, :].T for blk in params])
    dw9 = jnp.stack([blk['branches'][1]['dw'][:, 0, :].T for blk in params])
    wt = jnp.stack([jnp.stack([blk['branches'][0]['pw_w'].T,
                               blk['branches'][1]['pw_w'].T])
                    for blk in params])
    bs = jnp.stack([blk['branches'][0]['pw_b'] + blk['branches'][1]['pw_b']
                    for blk in params])
    g = jnp.stack([blk['ln_g'] for blk in params])
    lb = jnp.stack([blk['ln_b'] for blk in params])
    return _run(x, mask, emb, dw5, dw9, wt, bs, g, lb)
